# baseline (speedup 1.0000x reference)
import functools

import jax
import jax.numpy as jnp
from jax import lax
from jax.experimental import pallas as pl
from jax.experimental.pallas import tpu as pltpu
from jax.experimental.pallas import tpu_sc as plsc

G = 2
V = 1024
D = 512
DIN = 512
TAU = 0.5

TB1 = 512
TB3 = 2048

SC_NC = 2
SC_NS = 16
SC_NW = SC_NC * SC_NS
SC_CHUNK = 128


def _stage1_body(*refs):
    x_ref, u_ref, w1_ref, b1_ref = refs[:4]
    w1p_ref, b1p_ref = refs[-2:]
    p_ref, idx_ref = refs[-4:-2]
    @pl.when(pl.program_id(0) == 0)
    def _():
        co = lax.broadcasted_iota(jnp.int32, (2 * SC_CHUNK, SC_CHUNK), 0)
        oo = lax.broadcasted_iota(jnp.int32, (2 * SC_CHUNK, SC_CHUNK), 1)
        for g in range(G):
            sel = (co == G * oo + g).astype(jnp.float32)
            for j in range(G * V // 256):
                src = slice(j * 256, (j + 1) * 256)
                dst = slice(g * V + j * 128, g * V + (j + 1) * 128)
                w1p_ref[:, dst] = jnp.dot(w1_ref[:, src], sel)
                b1p_ref[:, dst] = jnp.dot(b1_ref[:, src], sel)

    xb = x_ref[...]
    h = jnp.maximum(jnp.dot(xb, w1p_ref[...]) + b1p_ref[...], 0.0)
    idx_rows = []
    for g in range(G):
        xg = -jnp.log(-jnp.log(u_ref[:, g, :])) + h[:, g * V:(g + 1) * V]
        e = jnp.exp(xg)
        s = jnp.sum(e, axis=1, keepdims=True)
        p_ref[:, g, :] = (e / s) * (1.0 / TAU)
        m = jnp.max(xg, axis=1, keepdims=True)
        io = lax.broadcasted_iota(jnp.int32, xg.shape, 1)
        idx = jnp.min(jnp.where(xg >= m, io, V), axis=1, keepdims=True) + g * V
        idx_rows.append(jnp.reshape(idx, (TB1 // 128, 128)))
    idx_ref[0] = jnp.concatenate(idx_rows, axis=0)


def _stage1(x2d, u3d, w1p, b1p, blk_lo, nblk, p_alias=None):
    bt = x2d.shape[0]
    in_specs = [
        pl.BlockSpec((TB1, DIN), lambda i, o=blk_lo: (i + o, 0)),
        pl.BlockSpec((TB1, G, V), lambda i, o=blk_lo: (i + o, 0, 0)),
        pl.BlockSpec((DIN, G * V), lambda i: (0, 0)),
        pl.BlockSpec((1, G * V), lambda i: (0, 0)),
    ]
    args = [x2d, u3d, w1p, b1p]
    aliases = {}
    if p_alias is not None:
        in_specs.append(pl.BlockSpec(memory_space=pl.ANY))
        args.append(p_alias)
        aliases = {4: 0}
    return pl.pallas_call(
        _stage1_body,
        grid=(nblk,),
        in_specs=in_specs,
        out_specs=[
            pl.BlockSpec((TB1, G, V), lambda i, o=blk_lo: (i + o, 0, 0)),
            pl.BlockSpec((1, G * TB1 // 128, 128), lambda i: (i, 0, 0)),
        ],
        out_shape=[
            jax.ShapeDtypeStruct((bt, G, V), jnp.float32),
            jax.ShapeDtypeStruct((nblk, G * TB1 // 128, 128), jnp.int32),
        ],
        scratch_shapes=[
            pltpu.VMEM((DIN, G * V), jnp.float32),
            pltpu.VMEM((1, G * V), jnp.float32),
        ],
        input_output_aliases=aliases,
    )(*args)


def _stage3_body(*refs):
    s_ref, w2_ref, b2_ref = refs[:3]
    q_ref = refs[-1]
    q_ref[...] = jnp.maximum(jnp.dot(s_ref[...], w2_ref[...]) + b2_ref[...], 0.0)


def _stage3(sub, W2, b2row, bt, blk_lo, nblk, q_alias=None):
    in_specs = [
        pl.BlockSpec((TB3, G * D), lambda i: (i, 0)),
        pl.BlockSpec((G * D, D), lambda i: (0, 0)),
        pl.BlockSpec((1, D), lambda i: (0, 0)),
    ]
    args = [sub, W2, b2row]
    aliases = {}
    if q_alias is not None:
        in_specs.append(pl.BlockSpec(memory_space=pl.ANY))
        args.append(q_alias)
        aliases = {3: 0}
    return pl.pallas_call(
        _stage3_body,
        grid=(nblk,),
        in_specs=in_specs,
        out_specs=pl.BlockSpec((TB3, D), lambda i, o=blk_lo: (i + o, 0)),
        out_shape=jax.ShapeDtypeStruct((bt, D), jnp.float32),
        input_output_aliases=aliases,
    )(*args)


def _sc_gather(table, idx3, ntok):
    cpb = TB1 // SC_CHUNK
    if (ntok // SC_NW) % SC_CHUNK == 0:
        tw, nw_active = ntok // SC_NW, SC_NW
    elif ntok % (2 * SC_CHUNK) == 0 and ntok // (2 * SC_CHUNK) <= SC_NW:
        tw, nw_active = 2 * SC_CHUNK, ntok // (2 * SC_CHUNK)
    else:
        tw, nw_active = ntok // SC_NW, SC_NW
    mesh = plsc.VectorSubcoreMesh(core_axis_name="c", subcore_axis_name="s")

    @functools.partial(
        pl.kernel,
        mesh=mesh,
        out_type=jax.ShapeDtypeStruct((ntok, G * D), jnp.float32),
        scratch_types=[
            pltpu.VMEM((2, G * cpb, SC_CHUNK), jnp.int32),
            pltpu.VMEM((SC_CHUNK, D), jnp.float32),
            pltpu.SemaphoreType.DMA,
        ],
    )
    def k(table_hbm, idx_hbm, out_hbm, idx_v, rows_v, sem):
        wid = lax.axis_index("s") * SC_NC + lax.axis_index("c")

        def work():
            ts = wid * tw
            ow = ts // TB1
            c0 = (ts % TB1) // SC_CHUNK
            if tw % SC_CHUNK == 0:
                pltpu.sync_copy(idx_hbm.at[ow], idx_v.at[0])
                for g in range(G):
                    for kk in range(tw // SC_CHUNK):
                        pltpu.async_copy(
                            table_hbm.at[idx_v.at[0, g * cpb + c0 + kk]],
                            rows_v, sem).wait()
                        pltpu.sync_copy(
                            rows_v,
                            out_hbm.at[pl.ds(ts + kk * SC_CHUNK, SC_CHUNK),
                                       pl.ds(g * D, D)])
            else:
                ow1 = (ts + tw - 1) // TB1
                pltpu.sync_copy(idx_hbm.at[ow], idx_v.at[0])
                pltpu.sync_copy(idx_hbm.at[ow1], idx_v.at[1])
                for g in range(G):
                    for kk in range(tw // 64):
                        t_u = ts + 64 * kk
                        lb = t_u // TB1 - ow
                        c_u = (t_u % TB1) // SC_CHUNK
                        lo = t_u % SC_CHUNK
                        pltpu.async_copy(
                            table_hbm.at[
                                idx_v.at[lb, g * cpb + c_u, pl.ds(lo, 64)]],
                            rows_v.at[pl.ds(0, 64)], sem).wait()
                        pltpu.sync_copy(
                            rows_v.at[pl.ds(0, 64)],
                            out_hbm.at[pl.ds(t_u, 64), pl.ds(g * D, D)])

        if nw_active < SC_NW:
            pl.when(wid < nw_active)(work)
        else:
            work()

    return k(table, idx3)


def kernel(data, u, W1, b1, W2, b2, code_book):
    B, T, _ = data.shape
    bt = B * T
    half = bt // 2
    x2d = data.reshape(bt, DIN)
    u3d = u.reshape(bt, G, V)
    w1p = W1
    b1p = b1.reshape(1, G * V)
    table = code_book.reshape(G * V, D)
    b2row = b2.reshape(1, D)

    chunks = [3 * bt // 4, bt // 4]

    p_buf, idxs, t0 = None, [], 0
    for ntok in chunks:
        p_buf, idx_c = _stage1(x2d, u3d, w1p, b1p, t0 // TB1, ntok // TB1,
                               p_alias=p_buf)
        idxs.append(idx_c)
        t0 += ntok

    subs = [_sc_gather(table, idx_c, ntok)
            for idx_c, ntok in zip(idxs, chunks)]

    q_buf, t0 = None, 0
    for sub, ntok in zip(subs, chunks):
        q_buf = _stage3(sub, W2, b2row, bt, t0 // TB3, ntok // TB3,
                        q_alias=q_buf)
        t0 += ntok

    return (p_buf.reshape(B, T, G, V), q_buf.reshape(B, T, D))


# speedup vs baseline: 1.0650x; 1.0650x over previous
import functools

import jax
import jax.numpy as jnp
from jax import lax
from jax.experimental import pallas as pl
from jax.experimental.pallas import tpu as pltpu
from jax.experimental.pallas import tpu_sc as plsc

G = 2
V = 1024
D = 512
DIN = 512
TAU = 0.5

TB1 = 512
TB3 = 2048

SC_NC = 2
SC_NS = 16
SC_NW = SC_NC * SC_NS
SC_CHUNK = 128


def _stage1_body(*refs):
    x_ref, u_ref, w1_ref, b1_ref = refs[:4]
    w1p_ref, b1p_ref = refs[-2:]
    p_ref, idx_ref = refs[-4:-2]
    @pl.when(pl.program_id(0) == 0)
    def _():
        co = lax.broadcasted_iota(jnp.int32, (2 * SC_CHUNK, SC_CHUNK), 0)
        oo = lax.broadcasted_iota(jnp.int32, (2 * SC_CHUNK, SC_CHUNK), 1)
        for g in range(G):
            sel = (co == G * oo + g).astype(jnp.float32)
            for j in range(G * V // 256):
                src = slice(j * 256, (j + 1) * 256)
                dst = slice(g * V + j * 128, g * V + (j + 1) * 128)
                w1p_ref[:, dst] = jnp.dot(w1_ref[:, src], sel)
                b1p_ref[:, dst] = jnp.dot(b1_ref[:, src], sel)

    xb = x_ref[...]
    h = jnp.maximum(jnp.dot(xb, w1p_ref[...]) + b1p_ref[...], 0.0)
    idx_rows = []
    for g in range(G):
        xg = -jnp.log(-jnp.log(u_ref[:, g, :])) + h[:, g * V:(g + 1) * V]
        e = jnp.exp(xg)
        s = jnp.sum(e, axis=1, keepdims=True)
        p_ref[:, g, :] = (e / s) * (1.0 / TAU)
        m = jnp.max(xg, axis=1, keepdims=True)
        io = lax.broadcasted_iota(jnp.int32, xg.shape, 1)
        idx = jnp.min(jnp.where(xg >= m, io, V), axis=1, keepdims=True) + g * V
        idx_rows.append(jnp.reshape(idx, (TB1 // 128, 128)))
    idx_ref[0] = jnp.concatenate(idx_rows, axis=0)


def _stage1(x2d, u3d, w1p, b1p, blk_lo, nblk, p_alias=None):
    bt = x2d.shape[0]
    in_specs = [
        pl.BlockSpec((TB1, DIN), lambda i, o=blk_lo: (i + o, 0)),
        pl.BlockSpec((TB1, G, V), lambda i, o=blk_lo: (i + o, 0, 0)),
        pl.BlockSpec((DIN, G * V), lambda i: (0, 0)),
        pl.BlockSpec((1, G * V), lambda i: (0, 0)),
    ]
    args = [x2d, u3d, w1p, b1p]
    aliases = {}
    if p_alias is not None:
        in_specs.append(pl.BlockSpec(memory_space=pl.ANY))
        args.append(p_alias)
        aliases = {4: 0}
    return pl.pallas_call(
        _stage1_body,
        grid=(nblk,),
        in_specs=in_specs,
        out_specs=[
            pl.BlockSpec((TB1, G, V), lambda i, o=blk_lo: (i + o, 0, 0)),
            pl.BlockSpec((1, G * TB1 // 128, 128), lambda i: (i, 0, 0)),
        ],
        out_shape=[
            jax.ShapeDtypeStruct((bt, G, V), jnp.float32),
            jax.ShapeDtypeStruct((nblk, G * TB1 // 128, 128), jnp.int32),
        ],
        scratch_shapes=[
            pltpu.VMEM((DIN, G * V), jnp.float32),
            pltpu.VMEM((1, G * V), jnp.float32),
        ],
        input_output_aliases=aliases,
    )(*args)


def _stage3_body(*refs):
    s_ref, w2_ref, b2_ref = refs[:3]
    q_ref = refs[-1]
    q_ref[...] = jnp.maximum(jnp.dot(s_ref[...], w2_ref[...]) + b2_ref[...], 0.0)


def _stage3(sub, W2, b2row, bt, blk_lo, nblk, q_alias=None):
    in_specs = [
        pl.BlockSpec((TB3, G * D), lambda i: (i, 0)),
        pl.BlockSpec((G * D, D), lambda i: (0, 0)),
        pl.BlockSpec((1, D), lambda i: (0, 0)),
    ]
    args = [sub, W2, b2row]
    aliases = {}
    if q_alias is not None:
        in_specs.append(pl.BlockSpec(memory_space=pl.ANY))
        args.append(q_alias)
        aliases = {3: 0}
    return pl.pallas_call(
        _stage3_body,
        grid=(nblk,),
        in_specs=in_specs,
        out_specs=pl.BlockSpec((TB3, D), lambda i, o=blk_lo: (i + o, 0)),
        out_shape=jax.ShapeDtypeStruct((bt, D), jnp.float32),
        input_output_aliases=aliases,
    )(*args)


def _sc_gather(table, idx3, ntok):
    tw = ntok // SC_NW
    cpb = TB1 // SC_CHUNK
    mesh = plsc.VectorSubcoreMesh(core_axis_name="c", subcore_axis_name="s")

    @functools.partial(
        pl.kernel,
        mesh=mesh,
        out_type=jax.ShapeDtypeStruct((ntok, G * D), jnp.float32),
        scratch_types=[
            pltpu.VMEM((2, G * cpb, SC_CHUNK), jnp.int32),
            pltpu.VMEM((SC_CHUNK, D), jnp.float32),
            pltpu.SemaphoreType.DMA,
        ],
    )
    def k(table_hbm, idx_hbm, out_hbm, idx_v, rows_v, sem):
        wid = lax.axis_index("s") * SC_NC + lax.axis_index("c")
        ts = wid * tw
        ow = ts // TB1
        c0 = (ts % TB1) // SC_CHUNK
        if tw % SC_CHUNK == 0:
            pltpu.sync_copy(idx_hbm.at[ow], idx_v.at[0])
            for g in range(G):
                for kk in range(tw // SC_CHUNK):
                    pltpu.async_copy(
                        table_hbm.at[idx_v.at[0, g * cpb + c0 + kk]], rows_v,
                        sem).wait()
                    pltpu.sync_copy(
                        rows_v,
                        out_hbm.at[pl.ds(ts + kk * SC_CHUNK, SC_CHUNK),
                                   pl.ds(g * D, D)])
        else:
            ow1 = (ts + tw - 1) // TB1
            pltpu.sync_copy(idx_hbm.at[ow], idx_v.at[0])
            pltpu.sync_copy(idx_hbm.at[ow1], idx_v.at[1])
            for g in range(G):
                for kk in range(tw // 64):
                    t_u = ts + 64 * kk
                    lb = t_u // TB1 - ow
                    c_u = (t_u % TB1) // SC_CHUNK
                    lo = t_u % SC_CHUNK
                    pltpu.async_copy(
                        table_hbm.at[idx_v.at[lb, g * cpb + c_u, pl.ds(lo, 64)]],
                        rows_v.at[pl.ds(0, 64)], sem).wait()
                    pltpu.sync_copy(
                        rows_v.at[pl.ds(0, 64)],
                        out_hbm.at[pl.ds(t_u, 64), pl.ds(g * D, D)])

    return k(table, idx3)


def kernel(data, u, W1, b1, W2, b2, code_book):
    B, T, _ = data.shape
    bt = B * T
    half = bt // 2
    x2d = data.reshape(bt, DIN)
    u3d = u.reshape(bt, G, V)
    w1p = W1
    b1p = b1.reshape(1, G * V)
    table = code_book.reshape(G * V, D)
    b2row = b2.reshape(1, D)

    chunks = [3 * bt // 4, bt // 4]

    p_buf, idxs, t0 = None, [], 0
    for ntok in chunks:
        p_buf, idx_c = _stage1(x2d, u3d, w1p, b1p, t0 // TB1, ntok // TB1,
                               p_alias=p_buf)
        idxs.append(idx_c)
        t0 += ntok

    subs = [_sc_gather(table, idx_c, ntok)
            for idx_c, ntok in zip(idxs, chunks)]

    q_buf, t0 = None, 0
    for sub, ntok in zip(subs, chunks):
        q_buf = _stage3(sub, W2, b2row, bt, t0 // TB3, ntok // TB3,
                        q_alias=q_buf)
        t0 += ntok

    return (p_buf.reshape(B, T, G, V), q_buf.reshape(B, T, D))
